# Initial kernel scaffold; baseline (speedup 1.0000x reference)
import jax
import jax.numpy as jnp
from jax.experimental import pallas as pl


def kernel(position_ids, pe):
    raise NotImplementedError("write your pallas kernel here")


# speedup vs baseline: 2.4656x; 2.4656x over previous
import jax
import jax.numpy as jnp
from jax import lax
from jax.experimental import pallas as pl
from jax.experimental.pallas import tpu as pltpu
from jax.experimental.pallas import tpu_sc as plsc

_NC = 2
_NS = 16
_NW = _NC * _NS
_CHUNK = 64


def _make_sc_gather(n_idx, d_model, dtype):
    per_w = n_idx // _NW
    n_chunks = per_w // _CHUNK
    mesh = plsc.VectorSubcoreMesh(core_axis_name="c", subcore_axis_name="s")

    def body(idx_hbm, table_hbm, out_hbm, idx_v, rows0, rows1,
             gsem0, gsem1, ssem0, ssem1):
        wid = lax.axis_index("s") * _NC + lax.axis_index("c")
        base = wid * per_w
        pltpu.sync_copy(idx_hbm.at[wid], idx_v)

        rows = (rows0, rows1)
        gsems = (gsem0, gsem1)
        ssems = (ssem0, ssem1)
        gets = [None] * n_chunks
        puts = [None] * n_chunks
        gets[0] = pltpu.async_copy(table_hbm.at[idx_v.at[0]], rows[0], gsems[0])
        if n_chunks > 1:
            gets[1] = pltpu.async_copy(table_hbm.at[idx_v.at[1]], rows[1], gsems[1])
        for j in range(n_chunks):
            b = j % 2
            gets[j].wait()
            puts[j] = pltpu.async_copy(
                rows[b], out_hbm.at[pl.ds(base + j * _CHUNK, _CHUNK)], ssems[b])
            if j + 2 < n_chunks:
                puts[j].wait()
                gets[j + 2] = pltpu.async_copy(
                    table_hbm.at[idx_v.at[j + 2]], rows[b], gsems[b])
        for j in range(max(n_chunks - 2, 0), n_chunks):
            puts[j].wait()

    return pl.kernel(
        body,
        mesh=mesh,
        out_type=jax.ShapeDtypeStruct((n_idx, d_model), dtype),
        scratch_types=[
            pltpu.VMEM((n_chunks, _CHUNK), jnp.int32),
            pltpu.VMEM((_CHUNK, d_model), dtype),
            pltpu.VMEM((_CHUNK, d_model), dtype),
            pltpu.SemaphoreType.DMA,
            pltpu.SemaphoreType.DMA,
            pltpu.SemaphoreType.DMA,
            pltpu.SemaphoreType.DMA,
        ],
    )


def kernel(position_ids, pe):
    b, s = position_ids.shape
    _, d = pe.shape
    n = b * s
    per_w = n // _NW
    idx = position_ids.reshape(_NW, per_w // _CHUNK, _CHUNK)
    out = _make_sc_gather(n, d, pe.dtype)(idx, pe)
    return out.reshape(b, s, d)
